# Initial kernel scaffold; baseline (speedup 1.0000x reference)
import jax
import jax.numpy as jnp
from jax.experimental import pallas as pl


def kernel(obs_img, obs_dir, prev_action, prev_reward, done, tile_table, color_table, action_table, dir_W, dir_b):
    raise NotImplementedError("write your pallas kernel here")


# speedup vs baseline: 9.5344x; 9.5344x over previous
import functools

import jax
import jax.numpy as jnp
from jax import lax
from jax.experimental import pallas as pl
from jax.experimental.pallas import tpu as pltpu
from jax.experimental.pallas import tpu_sc as plsc

F32 = jnp.float32
I32 = jnp.int32

_B, _T, _H, _W = 1024, 50, 5, 5
_N = _B * _T
_P = _H * _W
_E = 32
_IMG = _P * _E
_TAIL = 34
_ROW = _IMG + _TAIL
_NW = 32
_RPW = _N // _NW
_NB = 80
_CHUNKS = _RPW // _NB
_G = _NB // 16


def _tc_setup_body(dir_ref, act_ref, rew_ref, done_ref, tile_ref, color_ref,
                   atab_ref, w_ref, b_ref, tail_ref, fused_ref):
    de = jnp.dot(dir_ref[...], w_ref[...], preferred_element_type=F32) + b_ref[...]
    a = act_ref[...]
    oh = (a == lax.broadcasted_iota(I32, (a.shape[0], 16), 1)).astype(F32)
    ae = jnp.dot(oh, atab_ref[...], preferred_element_type=F32)
    tail_ref[:, 0:16] = de
    tail_ref[:, 16:32] = ae
    tail_ref[:, 32:33] = rew_ref[...]
    tail_ref[:, 33:34] = done_ref[...]
    r = lax.broadcasted_iota(I32, (256, 16), 0)
    k = lax.broadcasted_iota(I32, (256, 16), 1)
    oht = ((r // 16) == k).astype(F32)
    ohc = ((r % 16) == k).astype(F32)
    fused_ref[:, 0:16] = jnp.dot(oht, tile_ref[...], preferred_element_type=F32)
    fused_ref[:, 16:32] = jnp.dot(ohc, color_ref[...], preferred_element_type=F32)


def _tc_setup(obs_dir, prev_action, prev_reward, done, tile_table, color_table,
              action_table, dir_W, dir_b):
    nblk = 1024
    grid = _N // nblk
    return pl.pallas_call(
        _tc_setup_body,
        grid=(grid,),
        in_specs=[
            pl.BlockSpec((nblk, 4), lambda i: (i, 0)),
            pl.BlockSpec((nblk, 1), lambda i: (i, 0)),
            pl.BlockSpec((nblk, 1), lambda i: (i, 0)),
            pl.BlockSpec((nblk, 1), lambda i: (i, 0)),
            pl.BlockSpec((16, 16), lambda i: (0, 0)),
            pl.BlockSpec((16, 16), lambda i: (0, 0)),
            pl.BlockSpec((16, 16), lambda i: (0, 0)),
            pl.BlockSpec((4, 16), lambda i: (0, 0)),
            pl.BlockSpec((1, 16), lambda i: (0, 0)),
        ],
        out_specs=[
            pl.BlockSpec((nblk, _TAIL), lambda i: (i, 0)),
            pl.BlockSpec((256, 32), lambda i: (0, 0)),
        ],
        out_shape=[
            jax.ShapeDtypeStruct((_N, _TAIL), F32),
            jax.ShapeDtypeStruct((256, 32), F32),
        ],
    )(obs_dir, prev_action, prev_reward, done, tile_table, color_table,
      action_table, dir_W, dir_b)


@functools.partial(
    pl.kernel,
    mesh=plsc.VectorSubcoreMesh(core_axis_name="c", subcore_axis_name="s"),
    compiler_params=pltpu.CompilerParams(use_tc_tiling_on_sc=False,
                                         needs_layout_passes=False),
    out_type=jax.ShapeDtypeStruct((_N, _ROW), F32),
    scratch_types=[
        pltpu.VMEM((_NB, 2 * _P), I32),
        pltpu.VMEM((_P, _NB), I32),
        pltpu.VMEM((_P, _NB, _E), F32),
        pltpu.VMEM((_NB, _TAIL), F32),
        pltpu.SemaphoreType.DMA,
        pltpu.SemaphoreType.DMA,
    ],
)
def _sc_encode(obs_hbm, fused_hbm, tail_hbm, out_hbm,
               obs_v, idx_v, img_v, tail_v, gsem, osem):
    wid = lax.axis_index("s") * 2 + lax.axis_index("c")
    base = wid * _RPW
    iota = lax.iota(I32, 16)

    def chunk_body(i, carry):
        r0 = base + i * _NB
        pltpu.sync_copy(obs_hbm.at[pl.ds(r0, _NB), :], obs_v)

        def build(k, c):
            p = k // _G
            g = k % _G
            rows = g * 16 + iota
            t = plsc.load_gather(obs_v, [rows, jnp.full((16,), 2 * p, I32)])
            cc = plsc.load_gather(obs_v, [rows, jnp.full((16,), 2 * p + 1, I32)])
            idx_v[p, pl.ds(g * 16, 16)] = t * 16 + cc
            return c

        lax.fori_loop(0, _P * _G, build, 0)

        def gath(p, c):
            pltpu.async_copy(fused_hbm.at[idx_v.at[p]], img_v.at[p], gsem).wait()
            pltpu.async_copy(img_v.at[p],
                             out_hbm.at[pl.ds(r0, _NB), pl.ds(_E * p, _E)],
                             osem).wait()
            return c

        lax.fori_loop(0, _P, gath, 0)

        pltpu.sync_copy(tail_hbm.at[pl.ds(r0, _NB), :], tail_v)
        pltpu.sync_copy(tail_v, out_hbm.at[pl.ds(r0, _NB), pl.ds(_IMG, _TAIL)])
        return carry

    lax.fori_loop(0, _CHUNKS, chunk_body, 0)


def kernel(obs_img, obs_dir, prev_action, prev_reward, done, tile_table,
           color_table, action_table, dir_W, dir_b):
    obs_flat = obs_img.reshape(_N, 2 * _P).astype(I32)
    tail, fused = _tc_setup(
        obs_dir.reshape(_N, 4), prev_action.reshape(_N, 1).astype(I32),
        prev_reward.reshape(_N, 1), done.reshape(_N, 1),
        tile_table, color_table, action_table, dir_W, dir_b.reshape(1, 16))
    out = _sc_encode(obs_flat, fused, tail)
    return out.reshape(_B, _T, _ROW)


# speedup vs baseline: 9.8861x; 1.0369x over previous
import functools

import jax
import jax.numpy as jnp
from jax import lax
from jax.experimental import pallas as pl
from jax.experimental.pallas import tpu as pltpu
from jax.experimental.pallas import tpu_sc as plsc

F32 = jnp.float32
I32 = jnp.int32

_B, _T, _H, _W = 1024, 50, 5, 5
_N = _B * _T
_P = _H * _W
_E = 32
_IMG = _P * _E
_TAIL = 34
_ROW = _IMG + _TAIL
_NW = 32
_RPW = _N // _NW
_NB = 80
_CHUNKS = _RPW // _NB
_G = _NB // 16


def _tc_setup_body(dir_ref, act_ref, rew_ref, done_ref, tile_ref, color_ref,
                   atab_ref, w_ref, b_ref, tail_ref, fused_ref):
    de = jnp.dot(dir_ref[...], w_ref[...], preferred_element_type=F32) + b_ref[...]
    a = act_ref[...]
    oh = (a == lax.broadcasted_iota(I32, (a.shape[0], 16), 1)).astype(F32)
    ae = jnp.dot(oh, atab_ref[...], preferred_element_type=F32)
    tail_ref[:, 0:16] = de
    tail_ref[:, 16:32] = ae
    tail_ref[:, 32:33] = rew_ref[...]
    tail_ref[:, 33:34] = done_ref[...]
    r = lax.broadcasted_iota(I32, (256, 16), 0)
    k = lax.broadcasted_iota(I32, (256, 16), 1)
    oht = ((r // 16) == k).astype(F32)
    ohc = ((r % 16) == k).astype(F32)
    fused_ref[:, 0:16] = jnp.dot(oht, tile_ref[...], preferred_element_type=F32)
    fused_ref[:, 16:32] = jnp.dot(ohc, color_ref[...], preferred_element_type=F32)


def _tc_setup(obs_dir, prev_action, prev_reward, done, tile_table, color_table,
              action_table, dir_W, dir_b):
    nblk = 1024
    grid = _N // nblk
    return pl.pallas_call(
        _tc_setup_body,
        grid=(grid,),
        in_specs=[
            pl.BlockSpec((nblk, 4), lambda i: (i, 0)),
            pl.BlockSpec((nblk, 1), lambda i: (i, 0)),
            pl.BlockSpec((nblk, 1), lambda i: (i, 0)),
            pl.BlockSpec((nblk, 1), lambda i: (i, 0)),
            pl.BlockSpec((16, 16), lambda i: (0, 0)),
            pl.BlockSpec((16, 16), lambda i: (0, 0)),
            pl.BlockSpec((16, 16), lambda i: (0, 0)),
            pl.BlockSpec((4, 16), lambda i: (0, 0)),
            pl.BlockSpec((1, 16), lambda i: (0, 0)),
        ],
        out_specs=[
            pl.BlockSpec((nblk, _TAIL), lambda i: (i, 0)),
            pl.BlockSpec((256, 32), lambda i: (0, 0)),
        ],
        out_shape=[
            jax.ShapeDtypeStruct((_N, _TAIL), F32),
            jax.ShapeDtypeStruct((256, 32), F32),
        ],
    )(obs_dir, prev_action, prev_reward, done, tile_table, color_table,
      action_table, dir_W, dir_b)


@functools.partial(
    pl.kernel,
    mesh=plsc.VectorSubcoreMesh(core_axis_name="c", subcore_axis_name="s"),
    compiler_params=pltpu.CompilerParams(use_tc_tiling_on_sc=False,
                                         needs_layout_passes=False),
    out_type=jax.ShapeDtypeStruct((_N, _ROW), F32),
    scratch_types=[
        pltpu.VMEM((_NB, 2 * _P), I32),
        pltpu.VMEM((_NB * _P,), I32),
        pltpu.VMEM((_NB * _P, _E), F32),
        pltpu.VMEM((_NB, _TAIL), F32),
        pltpu.SemaphoreType.DMA,
        pltpu.SemaphoreType.DMA,
    ],
)
def _sc_encode(obs_hbm, fused_hbm, tail_hbm, out_hbm,
               obs_v, idx_v, img_v, tail_v, gsem, osem):
    wid = lax.axis_index("s") * 2 + lax.axis_index("c")
    base = wid * _RPW
    iota = lax.iota(I32, 16)

    def chunk_body(i, carry):
        r0 = base + i * _NB
        pltpu.sync_copy(obs_hbm.at[pl.ds(r0, _NB), :], obs_v)

        def build(k, c):
            p = k // _G
            g = k % _G
            rows = g * 16 + iota
            t = plsc.load_gather(obs_v, [rows, jnp.full((16,), 2 * p, I32)])
            cc = plsc.load_gather(obs_v, [rows, jnp.full((16,), 2 * p + 1, I32)])
            plsc.store_scatter(idx_v, [p * _NB + rows], t * 16 + cc)
            return c

        lax.fori_loop(0, _P * _G, build, 0)

        pltpu.async_copy(fused_hbm.at[idx_v], img_v, gsem).wait()
        cps = [pltpu.async_copy(img_v.at[pl.ds(_NB * p, _NB)],
                                out_hbm.at[pl.ds(r0, _NB), pl.ds(_E * p, _E)],
                                osem)
               for p in range(_P)]
        for cp in cps:
            cp.wait()

        pltpu.sync_copy(tail_hbm.at[pl.ds(r0, _NB), :], tail_v)
        pltpu.sync_copy(tail_v, out_hbm.at[pl.ds(r0, _NB), pl.ds(_IMG, _TAIL)])
        return carry

    lax.fori_loop(0, _CHUNKS, chunk_body, 0)


def kernel(obs_img, obs_dir, prev_action, prev_reward, done, tile_table,
           color_table, action_table, dir_W, dir_b):
    obs_flat = obs_img.reshape(_N, 2 * _P).astype(I32)
    tail, fused = _tc_setup(
        obs_dir.reshape(_N, 4), prev_action.reshape(_N, 1).astype(I32),
        prev_reward.reshape(_N, 1), done.reshape(_N, 1),
        tile_table, color_table, action_table, dir_W, dir_b.reshape(1, 16))
    out = _sc_encode(obs_flat, fused, tail)
    return out.reshape(_B, _T, _ROW)


# speedup vs baseline: 12.2255x; 1.2366x over previous
import functools

import jax
import jax.numpy as jnp
from jax import lax
from jax.experimental import pallas as pl
from jax.experimental.pallas import tpu as pltpu
from jax.experimental.pallas import tpu_sc as plsc

F32 = jnp.float32
I32 = jnp.int32

_B, _T, _H, _W = 1024, 50, 5, 5
_N = _B * _T
_P = _H * _W
_E = 32
_IMG = _P * _E
_TAIL = 34
_ROW = _IMG + _TAIL
_NW = 32
_RPW = _N // _NW
_NB = 80
_CHUNKS = _RPW // _NB
_G = _NB // 16


def _tc_setup_body(dir_ref, act_ref, rew_ref, done_ref, tile_ref, color_ref,
                   atab_ref, w_ref, b_ref, tail_ref, fused_ref):
    de = jnp.dot(dir_ref[...], w_ref[...], preferred_element_type=F32) + b_ref[...]
    a = act_ref[...]
    oh = (a == lax.broadcasted_iota(I32, (a.shape[0], 16), 1)).astype(F32)
    ae = jnp.dot(oh, atab_ref[...], preferred_element_type=F32)
    tail_ref[:, 0:16] = de
    tail_ref[:, 16:32] = ae
    tail_ref[:, 32:33] = rew_ref[...]
    tail_ref[:, 33:34] = done_ref[...]
    r = lax.broadcasted_iota(I32, (256, 16), 0)
    k = lax.broadcasted_iota(I32, (256, 16), 1)
    oht = ((r // 16) == k).astype(F32)
    ohc = ((r % 16) == k).astype(F32)
    fused_ref[:, 0:16] = jnp.dot(oht, tile_ref[...], preferred_element_type=F32)
    fused_ref[:, 16:32] = jnp.dot(ohc, color_ref[...], preferred_element_type=F32)


def _tc_setup(obs_dir, prev_action, prev_reward, done, tile_table, color_table,
              action_table, dir_W, dir_b):
    nblk = 1024
    grid = _N // nblk
    return pl.pallas_call(
        _tc_setup_body,
        grid=(grid,),
        in_specs=[
            pl.BlockSpec((nblk, 4), lambda i: (i, 0)),
            pl.BlockSpec((nblk, 1), lambda i: (i, 0)),
            pl.BlockSpec((nblk, 1), lambda i: (i, 0)),
            pl.BlockSpec((nblk, 1), lambda i: (i, 0)),
            pl.BlockSpec((16, 16), lambda i: (0, 0)),
            pl.BlockSpec((16, 16), lambda i: (0, 0)),
            pl.BlockSpec((16, 16), lambda i: (0, 0)),
            pl.BlockSpec((4, 16), lambda i: (0, 0)),
            pl.BlockSpec((1, 16), lambda i: (0, 0)),
        ],
        out_specs=[
            pl.BlockSpec((nblk, _TAIL), lambda i: (i, 0)),
            pl.BlockSpec((256, 32), lambda i: (0, 0)),
        ],
        out_shape=[
            jax.ShapeDtypeStruct((_N, _TAIL), F32),
            jax.ShapeDtypeStruct((256, 32), F32),
        ],
    )(obs_dir, prev_action, prev_reward, done, tile_table, color_table,
      action_table, dir_W, dir_b)


@functools.partial(
    pl.kernel,
    mesh=plsc.VectorSubcoreMesh(core_axis_name="c", subcore_axis_name="s"),
    compiler_params=pltpu.CompilerParams(use_tc_tiling_on_sc=False,
                                         needs_layout_passes=False),
    out_type=jax.ShapeDtypeStruct((_N, _ROW), F32),
    scratch_types=[
        pltpu.VMEM((256 * _E,), F32),
        pltpu.VMEM((_NB, 2 * _P), I32),
        pltpu.VMEM((_NB, _ROW), F32),
        pltpu.SemaphoreType.DMA,
    ],
)
def _sc_encode(obs_hbm, fused_hbm, tail_hbm, out_hbm,
               fused_v, obs_v, rowbuf, osem):
    wid = lax.axis_index("s") * 2 + lax.axis_index("c")
    base = wid * _RPW
    pltpu.sync_copy(fused_hbm, fused_v)

    def chunk_body(i, carry):
        r0 = base + i * _NB
        pltpu.sync_copy(obs_hbm.at[pl.ds(r0, _NB), :], obs_v)
        pltpu.sync_copy(tail_hbm.at[pl.ds(r0, _NB), :],
                        rowbuf.at[:, pl.ds(_IMG, _TAIL)])

        iota = lax.iota(I32, 16)
        mask9 = iota < 9

        def assemble(r, c):
            rr = jnp.full((16,), r, I32)
            ta = plsc.load_gather(obs_v, [rr, 2 * iota])
            ca = plsc.load_gather(obs_v, [rr, 2 * iota + 1])
            tb = plsc.load_gather(obs_v, [rr, 32 + 2 * iota], mask=mask9)
            cb = plsc.load_gather(obs_v, [rr, 33 + 2 * iota], mask=mask9)
            iva = (ta * 16 + ca) * _E
            ivb = (tb * 16 + cb) * _E
            for p in range(_P):
                off = iva[p] if p < 16 else ivb[p - 16]
                rowbuf[r, pl.ds(_E * p, 16)] = fused_v[pl.ds(off, 16)]
                rowbuf[r, pl.ds(_E * p + 16, 16)] = fused_v[pl.ds(off + 16, 16)]
            return c

        lax.fori_loop(0, _NB, assemble, 0)
        pltpu.sync_copy(rowbuf, out_hbm.at[pl.ds(r0, _NB), :])
        return carry

    lax.fori_loop(0, _CHUNKS, chunk_body, 0)


def kernel(obs_img, obs_dir, prev_action, prev_reward, done, tile_table,
           color_table, action_table, dir_W, dir_b):
    obs_flat = obs_img.reshape(_N, 2 * _P).astype(I32)
    tail, fused = _tc_setup(
        obs_dir.reshape(_N, 4), prev_action.reshape(_N, 1).astype(I32),
        prev_reward.reshape(_N, 1), done.reshape(_N, 1),
        tile_table, color_table, action_table, dir_W, dir_b.reshape(1, 16))
    out = _sc_encode(obs_flat, fused.reshape(-1), tail)
    return out.reshape(_B, _T, _ROW)


# speedup vs baseline: 17.3711x; 1.4209x over previous
import functools

import jax
import jax.numpy as jnp
from jax import lax
from jax.experimental import pallas as pl
from jax.experimental.pallas import tpu as pltpu
from jax.experimental.pallas import tpu_sc as plsc

F32 = jnp.float32
I32 = jnp.int32

_B, _T = 1024, 50
_P = 25
_E = 32
_IMG = _P * _E
_TAIL = 34
_ROW = _IMG + _TAIL
_NW = 32
_BPW = _B // _NW
_BB = 64
_M = _BB * _T


def _tc_setup_body(obs_ref, dir_ref, act_ref, rew_ref, done_ref, tile_ref,
                   color_ref, atab_ref, w_ref, b_ref, selt_ref, selc_ref,
                   pair_ref, tail_ref, fused_ref):
    x = obs_ref[...].reshape(_M, 2 * _P).astype(F32)
    t = jnp.dot(x, selt_ref[...], preferred_element_type=F32)
    c = jnp.dot(x, selc_ref[...], preferred_element_type=F32)
    pair_ref[...] = (t * 16.0 + c).astype(I32).reshape(_BB, _T, _P)

    de = jnp.dot(dir_ref[...].reshape(_M, 4), w_ref[...],
                 preferred_element_type=F32) + b_ref[...]
    a = act_ref[...].reshape(_M, 1)
    oh = (a == lax.broadcasted_iota(I32, (_M, 16), 1)).astype(F32)
    ae = jnp.dot(oh, atab_ref[...], preferred_element_type=F32)
    tail_ref[:, :, 0:16] = de.reshape(_BB, _T, 16)
    tail_ref[:, :, 16:32] = ae.reshape(_BB, _T, 16)
    tail_ref[:, :, 32:33] = rew_ref[...]
    tail_ref[:, :, 33:34] = done_ref[...]

    r = lax.broadcasted_iota(I32, (256, 16), 0)
    k = lax.broadcasted_iota(I32, (256, 16), 1)
    oht = ((r // 16) == k).astype(F32)
    ohc = ((r % 16) == k).astype(F32)
    fused_ref[:, 0:16] = jnp.dot(oht, tile_ref[...], preferred_element_type=F32)
    fused_ref[:, 16:32] = jnp.dot(ohc, color_ref[...], preferred_element_type=F32)


def _tc_setup(obs3, obs_dir, prev_action, prev_reward, done, tile_table,
              color_table, action_table, dir_W, dir_b, selt, selc):
    grid = _B // _BB
    bspec3 = lambda w: pl.BlockSpec((_BB, _T, w), lambda i: (i, 0, 0))
    fullspec = lambda s: pl.BlockSpec(s, lambda i: tuple(0 for _ in s))
    return pl.pallas_call(
        _tc_setup_body,
        grid=(grid,),
        in_specs=[
            bspec3(2 * _P),
            bspec3(4),
            bspec3(1),
            bspec3(1),
            bspec3(1),
            fullspec((16, 16)),
            fullspec((16, 16)),
            fullspec((16, 16)),
            fullspec((4, 16)),
            fullspec((1, 16)),
            fullspec((2 * _P, _P)),
            fullspec((2 * _P, _P)),
        ],
        out_specs=[
            bspec3(_P),
            bspec3(_TAIL),
            fullspec((256, 32)),
        ],
        out_shape=[
            jax.ShapeDtypeStruct((_B, _T, _P), I32),
            jax.ShapeDtypeStruct((_B, _T, _TAIL), F32),
            jax.ShapeDtypeStruct((256, 32), F32),
        ],
    )(obs3, obs_dir, prev_action, prev_reward, done, tile_table, color_table,
      action_table, dir_W, dir_b, selt, selc)


@functools.partial(
    pl.kernel,
    mesh=plsc.VectorSubcoreMesh(core_axis_name="c", subcore_axis_name="s"),
    compiler_params=pltpu.CompilerParams(use_tc_tiling_on_sc=True,
                                         needs_layout_passes=False),
    out_type=jax.ShapeDtypeStruct((_B, _T, _ROW), F32),
    scratch_types=[
        pltpu.VMEM((256 * _E,), F32),
        pltpu.VMEM((_T, _P), I32),
        pltpu.VMEM((_T, _TAIL), F32),
        pltpu.VMEM((_T, _ROW), F32),
        pltpu.SemaphoreType.DMA,
    ],
)
def _sc_encode(pair_hbm, fused_hbm, tail_hbm, out_hbm,
               fused_v, pair_v, tail_v, rowbuf, osem):
    wid = lax.axis_index("s") * 2 + lax.axis_index("c")
    b0 = wid * _BPW
    pltpu.sync_copy(fused_hbm, fused_v)

    def batch_body(i, carry):
        b = b0 + i
        pltpu.sync_copy(pair_hbm.at[b], pair_v)
        pltpu.sync_copy(tail_hbm.at[b], tail_v)

        def assemble(r, c):
            va = pair_v[r, pl.ds(0, 16)] * _E
            vb = pair_v[r, pl.ds(9, 16)] * _E
            for p in range(_P):
                off = va[p] if p < 16 else vb[p - 9]
                rowbuf[r, pl.ds(_E * p, 16)] = fused_v[pl.ds(off, 16)]
                rowbuf[r, pl.ds(_E * p + 16, 16)] = fused_v[pl.ds(off + 16, 16)]
            rowbuf[r, pl.ds(_IMG, 16)] = tail_v[r, pl.ds(0, 16)]
            rowbuf[r, pl.ds(_IMG + 16, 16)] = tail_v[r, pl.ds(16, 16)]
            rowbuf[r, pl.ds(_IMG + 18, 16)] = tail_v[r, pl.ds(18, 16)]
            return c

        lax.fori_loop(0, _T, assemble, 0)
        pltpu.sync_copy(rowbuf, out_hbm.at[b])
        return carry

    lax.fori_loop(0, _BPW, batch_body, 0)


def kernel(obs_img, obs_dir, prev_action, prev_reward, done, tile_table,
           color_table, action_table, dir_W, dir_b):
    obs3 = obs_img.reshape(_B, _T, 2 * _P).astype(I32)
    ii = jnp.arange(2 * _P)[:, None]
    jj = jnp.arange(_P)[None, :]
    selt = (ii == 2 * jj).astype(F32)
    selc = (ii == 2 * jj + 1).astype(F32)
    pair, tail, fused = _tc_setup(
        obs3, obs_dir, prev_action.reshape(_B, _T, 1).astype(I32),
        prev_reward.reshape(_B, _T, 1), done.reshape(_B, _T, 1),
        tile_table, color_table, action_table, dir_W, dir_b.reshape(1, 16),
        selt, selc)
    return _sc_encode(pair, fused.reshape(-1), tail)


# speedup vs baseline: 20.0555x; 1.1545x over previous
import functools

import jax
import jax.numpy as jnp
from jax import lax
from jax.experimental import pallas as pl
from jax.experimental.pallas import tpu as pltpu
from jax.experimental.pallas import tpu_sc as plsc

F32 = jnp.float32
I32 = jnp.int32

_B, _T = 1024, 50
_P = 25
_E = 32
_IMG = _P * _E
_TAIL = 34
_ROW = _IMG + _TAIL
_CW = 64
_NW = 32
_BPW = _B // _NW
_BB = 64
_M = _BB * _T


def _tc_setup_body(obs_ref, dir_ref, act_ref, rew_ref, done_ref, tile_ref,
                   color_ref, atab_ref, w_ref, b_ref, selt_ref, selc_ref,
                   combo_ref, fused_ref):
    x = obs_ref[...].reshape(_M, 2 * _P).astype(F32)
    t = jnp.dot(x, selt_ref[...], preferred_element_type=F32)
    c = jnp.dot(x, selc_ref[...], preferred_element_type=F32)
    combo_ref[:, :, 0:_P] = (t * 16.0 + c).reshape(_BB, _T, _P)

    de = jnp.dot(dir_ref[...].reshape(_M, 4), w_ref[...],
                 preferred_element_type=F32) + b_ref[...]
    a = act_ref[...].reshape(_M, 1)
    oh = (a == lax.broadcasted_iota(I32, (_M, 16), 1)).astype(F32)
    ae = jnp.dot(oh, atab_ref[...], preferred_element_type=F32)
    combo_ref[:, :, _P:_P + 16] = de.reshape(_BB, _T, 16)
    combo_ref[:, :, _P + 16:_P + 32] = ae.reshape(_BB, _T, 16)
    combo_ref[:, :, _P + 32:_P + 33] = rew_ref[...]
    combo_ref[:, :, _P + 33:_P + 34] = done_ref[...]
    combo_ref[:, :, _P + 34:] = jnp.zeros((_BB, _T, _CW - _P - _TAIL), F32)

    r = lax.broadcasted_iota(I32, (256, 16), 0)
    k = lax.broadcasted_iota(I32, (256, 16), 1)
    oht = ((r // 16) == k).astype(F32)
    ohc = ((r % 16) == k).astype(F32)
    fused_ref[:, 0:16] = jnp.dot(oht, tile_ref[...], preferred_element_type=F32)
    fused_ref[:, 16:32] = jnp.dot(ohc, color_ref[...], preferred_element_type=F32)


def _tc_setup(obs3, obs_dir, prev_action, prev_reward, done, tile_table,
              color_table, action_table, dir_W, dir_b, selt, selc):
    grid = _B // _BB
    bspec3 = lambda w: pl.BlockSpec((_BB, _T, w), lambda i: (i, 0, 0))
    fullspec = lambda s: pl.BlockSpec(s, lambda i: tuple(0 for _ in s))
    return pl.pallas_call(
        _tc_setup_body,
        grid=(grid,),
        in_specs=[
            bspec3(2 * _P),
            bspec3(4),
            bspec3(1),
            bspec3(1),
            bspec3(1),
            fullspec((16, 16)),
            fullspec((16, 16)),
            fullspec((16, 16)),
            fullspec((4, 16)),
            fullspec((1, 16)),
            fullspec((2 * _P, _P)),
            fullspec((2 * _P, _P)),
        ],
        out_specs=[
            bspec3(_CW),
            fullspec((256, 32)),
        ],
        out_shape=[
            jax.ShapeDtypeStruct((_B, _T, _CW), F32),
            jax.ShapeDtypeStruct((256, 32), F32),
        ],
    )(obs3, obs_dir, prev_action, prev_reward, done, tile_table, color_table,
      action_table, dir_W, dir_b, selt, selc)


@functools.partial(
    pl.kernel,
    mesh=plsc.VectorSubcoreMesh(core_axis_name="c", subcore_axis_name="s"),
    compiler_params=pltpu.CompilerParams(use_tc_tiling_on_sc=True,
                                         needs_layout_passes=False),
    out_type=jax.ShapeDtypeStruct((_B, _T, _ROW), F32),
    scratch_types=[
        pltpu.VMEM((256 * _E,), F32),
        pltpu.VMEM((_T, _CW), F32),
        pltpu.VMEM((_T, _CW), F32),
        pltpu.VMEM((_T, _ROW), F32),
        pltpu.VMEM((_T, _ROW), F32),
        pltpu.SemaphoreType.DMA,
        pltpu.SemaphoreType.DMA,
    ],
)
def _sc_encode(combo_hbm, fused_hbm, out_hbm,
               fused_v, combo0, combo1, row0, row1, isem, osem):
    wid = lax.axis_index("s") * 2 + lax.axis_index("c")
    b0 = wid * _BPW
    pltpu.sync_copy(fused_hbm, fused_v)
    pltpu.async_copy(combo_hbm.at[b0], combo0, isem)

    def assemble(combo_v, rowbuf):
        def body(r, c):
            pv = combo_v[r, pl.ds(0, 16)].astype(I32) * _E
            qv = combo_v[r, pl.ds(9, 16)].astype(I32) * _E
            for p in range(_P):
                off = pv[p] if p < 16 else qv[p - 9]
                rowbuf[r, pl.ds(_E * p, 16)] = fused_v[pl.ds(off, 16)]
                rowbuf[r, pl.ds(_E * p + 16, 16)] = fused_v[pl.ds(off + 16, 16)]
            rowbuf[r, pl.ds(_IMG, 16)] = combo_v[r, pl.ds(_P, 16)]
            rowbuf[r, pl.ds(_IMG + 16, 16)] = combo_v[r, pl.ds(_P + 16, 16)]
            rowbuf[r, pl.ds(_IMG + 18, 16)] = combo_v[r, pl.ds(_P + 18, 16)]
            return c

        lax.fori_loop(0, _T, body, 0)

    def stage(j, b, combo_v, combo_n, rowbuf):
        pltpu.make_async_copy(combo_hbm.at[b], combo_v, isem).wait()

        @pl.when(b + 1 < b0 + _BPW)
        def _():
            pltpu.async_copy(combo_hbm.at[b + 1], combo_n, isem)

        @pl.when(j > 1)
        def _():
            pltpu.make_async_copy(rowbuf, out_hbm.at[b], osem).wait()

        assemble(combo_v, rowbuf)
        pltpu.async_copy(rowbuf, out_hbm.at[b], osem)

    def pair_body(j, carry):
        b = b0 + 2 * j
        stage(2 * j, b, combo0, combo1, row0)
        stage(2 * j + 1, b + 1, combo1, combo0, row1)
        return carry

    lax.fori_loop(0, _BPW // 2, pair_body, 0)
    pltpu.make_async_copy(row0, out_hbm.at[b0], osem).wait()
    pltpu.make_async_copy(row1, out_hbm.at[b0], osem).wait()


def kernel(obs_img, obs_dir, prev_action, prev_reward, done, tile_table,
           color_table, action_table, dir_W, dir_b):
    obs3 = obs_img.reshape(_B, _T, 2 * _P).astype(I32)
    ii = jnp.arange(2 * _P)[:, None]
    jj = jnp.arange(_P)[None, :]
    selt = (ii == 2 * jj).astype(F32)
    selc = (ii == 2 * jj + 1).astype(F32)
    combo, fused = _tc_setup(
        obs3, obs_dir, prev_action.reshape(_B, _T, 1).astype(I32),
        prev_reward.reshape(_B, _T, 1), done.reshape(_B, _T, 1),
        tile_table, color_table, action_table, dir_W, dir_b.reshape(1, 16),
        selt, selc)
    return _sc_encode(combo, fused.reshape(-1))
